# baseline (speedup 1.0000x reference)
import functools

import jax
import jax.numpy as jnp
from jax import lax
from jax.experimental import pallas as pl
from jax.experimental.pallas import tpu as pltpu
from jax.experimental.pallas import tpu_sc as plsc

_BATCH = 16384
_EMB = 32
_NC = 2
_NS = 16
_NW = _NC * _NS
_BPW = _BATCH // _NW
_CH = 128
_L = 16

_ROWS = 1000000
_TL = 2048

_BT = 2048


def _tr_body(in_ref, eye_ref, out_ref):
    dn = (((0,), (0,)), ((), ()))
    for a in range(4):
        xs = in_ref[:, 512 * a:512 * (a + 1)]
        out_ref[:, 32 * a:32 * (a + 1)] = lax.dot_general(
            xs, eye_ref[...], dn, preferred_element_type=jnp.float32)


def _tc_transpose(tab_t, eye):
    ngrid = (_ROWS + _TL - 1) // _TL
    return pl.pallas_call(
        _tr_body,
        grid=(ngrid,),
        in_specs=[pl.BlockSpec((_EMB, _TL), lambda i: (0, i)),
                  pl.BlockSpec((_EMB, _EMB), lambda i: (0, 0))],
        out_specs=pl.BlockSpec((_TL // 4, 128), lambda i: (i, 0)),
        out_shape=jax.ShapeDtypeStruct((ngrid * (_TL // 4), 128),
                                       jnp.float32),
    )(tab_t, eye)


def _sc_gather_one(idx, table):
    mesh = plsc.VectorSubcoreMesh(core_axis_name="c", subcore_axis_name="s")

    @functools.partial(
        pl.kernel,
        out_type=jax.ShapeDtypeStruct((_BATCH // 4, 128), jnp.float32),
        mesh=mesh,
        compiler_params=pltpu.CompilerParams(use_tc_tiling_on_sc=True,
                                             needs_layout_passes=False),
        scratch_types=[
            pltpu.VMEM((_BPW,), jnp.int32),
            pltpu.VMEM((_BPW,), jnp.int32),
            pltpu.VMEM((_CH, 128), jnp.float32),
            pltpu.VMEM((_CH, 128), jnp.float32),
            pltpu.VMEM((_BPW // 4, 128), jnp.float32),
            pltpu.SemaphoreType.DMA,
            pltpu.SemaphoreType.DMA,
        ],
    )
    def gather(idx_hbm, tab_hbm, out_hbm,
               idx_v, vrow_v, slab_a, slab_b, out_v, sem_a, sem_b):
        wid = lax.axis_index("s") * _NC + lax.axis_index("c")
        base = pl.multiple_of(wid * _BPW, _BPW)
        base4 = pl.multiple_of(wid * (_BPW // 4), _BPW // 4)
        pltpu.sync_copy(idx_hbm.at[pl.ds(base, _BPW)], idx_v)

        def vrows(k, carry):
            s = pl.ds(k * _L, _L)
            r = idx_v[s]
            vrow_v[s] = lax.shift_left(
                lax.shift_right_logical(r, 11), 9) + lax.bitwise_and(r, 511)
            return carry

        lax.fori_loop(0, _BPW // _L, vrows, 0)

        def fire_chunk(c, slab_v, sem):
            s = pl.ds(pl.multiple_of(c * _CH, _CH), _CH)
            return pltpu.async_copy(tab_hbm.at[vrow_v.at[s]], slab_v, sem)

        def extract(slab_v, c):
            for g in range(_CH // _L):
                lrow = lax.iota(jnp.int32, _L) + g * _L
                off = pl.multiple_of(c * _CH + g * _L, _L)
                col0 = lax.bitwise_and(
                    lax.shift_right_logical(idx_v[pl.ds(off, _L)], 9),
                    3) * _EMB
                grow = lrow + c * _CH
                drow = lax.shift_right_logical(grow, 2)
                dcol0 = lax.bitwise_and(grow, 3) * _EMB
                for j in range(_EMB):
                    vals = plsc.load_gather(slab_v, [lrow, col0 + j])
                    plsc.store_scatter(out_v, [drow, dcol0 + j], vals)

        nchunks = _BPW // _CH
        fire_chunk(0, slab_a, sem_a)

        def body(k, _):
            ca = 2 * k
            fire_chunk(ca + 1, slab_b, sem_b)
            pltpu.make_async_copy(
                tab_hbm.at[pl.ds(0, _CH)], slab_a, sem_a).wait()
            extract(slab_a, ca)

            @pl.when(k < nchunks // 2 - 1)
            def _():
                fire_chunk(ca + 2, slab_a, sem_a)

            pltpu.make_async_copy(
                tab_hbm.at[pl.ds(0, _CH)], slab_b, sem_b).wait()
            extract(slab_b, ca + 1)
            return 0

        lax.fori_loop(0, nchunks // 2, body, 0)

        pltpu.sync_copy(out_v, out_hbm.at[pl.ds(base4, _BPW // 4)])

    return gather(idx, table)


def _mlp_body(u_ref, i_ref, w1u_ref, w1i_ref, b1_ref, w2_ref, b2_ref,
              w3_ref, b3_ref, o_ref):
    dn = (((1,), (1,)), ((), ()))
    x1 = lax.dot_general(u_ref[...], w1u_ref[...], dn,
                         preferred_element_type=jnp.float32)
    x1 = x1 + lax.dot_general(i_ref[...], w1i_ref[...], dn,
                              preferred_element_type=jnp.float32)
    x1 = jnp.maximum(x1 + b1_ref[...], 0.0)
    x2 = lax.dot_general(x1, w2_ref[...], dn,
                         preferred_element_type=jnp.float32)
    x2 = jnp.maximum(x2 + b2_ref[...], 0.0)
    z = jnp.sum(x2 * w3_ref[...], axis=1, keepdims=True)
    z = z + b3_ref[0]
    o_ref[...] = 1.0 / (1.0 + jnp.exp(-z))


def _tc_mlp(u_emb, i_emb, W1u, W1i, b1r, W2, b2r, W3, b3):
    grid = (_BATCH // _BT,)
    full = lambda shape: pl.BlockSpec(shape, lambda i: (0, 0))
    return pl.pallas_call(
        _mlp_body,
        grid=grid,
        in_specs=[
            pl.BlockSpec((_BT, _EMB), lambda i: (i, 0)),
            pl.BlockSpec((_BT, _EMB), lambda i: (i, 0)),
            full(W1u.shape),
            full(W1i.shape),
            full(b1r.shape),
            full(W2.shape),
            full(b2r.shape),
            full(W3.shape),
            pl.BlockSpec(memory_space=pltpu.SMEM),
        ],
        out_specs=pl.BlockSpec((_BT, 1), lambda i: (i, 0)),
        out_shape=jax.ShapeDtypeStruct((_BATCH, 1), jnp.float32),
    )(u_emb, i_emb, W1u, W1i, b1r, W2, b2r, W3, b3)


def kernel(user_idx, item_idx, user_table, item_table, W1, b1, W2, b2, W3, b3):
    uidx = user_idx.astype(jnp.int32)
    iidx = item_idx.astype(jnp.int32)
    eye = jnp.eye(_EMB, dtype=jnp.float32)
    u128 = _tc_transpose(user_table.T, eye)
    i128 = _tc_transpose(item_table.T, eye)
    u_out = _sc_gather_one(uidx, u128)
    i_out = _sc_gather_one(iidx, i128)
    u_emb = u_out.reshape(_BATCH, _EMB)
    i_emb = i_out.reshape(_BATCH, _EMB)
    W1u = W1[:, :_EMB]
    W1i = W1[:, _EMB:]
    return _tc_mlp(u_emb, i_emb, W1u, W1i,
                   b1.reshape(1, -1), W2, b2.reshape(1, -1),
                   W3, b3)


# speedup vs baseline: 1.5993x; 1.5993x over previous
import functools

import jax
import jax.numpy as jnp
from jax import lax
from jax.experimental import pallas as pl
from jax.experimental.pallas import tpu as pltpu
from jax.experimental.pallas import tpu_sc as plsc

_BATCH = 16384
_EMB = 32
_NC = 2
_NS = 16
_NW = _NC * _NS
_BPW = _BATCH // _NW
_CH = 128
_L = 16

_ROWS = 1000000
_TL = 8192
_TQ = _TL // 4

_BT = 2048


def _tr_body(in_ref, out_ref):
    t = jnp.transpose(in_ref[...], (1, 0))
    for a in range(4):
        out_ref[:, 32 * a:32 * (a + 1)] = lax.slice(
            t, (_TQ * a, 0), (_TQ * (a + 1), _EMB))


def _tc_transpose(tab_t):
    ngrid = (_ROWS + _TL - 1) // _TL
    return pl.pallas_call(
        _tr_body,
        grid=(ngrid,),
        in_specs=[pl.BlockSpec((_EMB, _TL), lambda i: (0, i))],
        out_specs=pl.BlockSpec((_TQ, 128), lambda i: (i, 0)),
        out_shape=jax.ShapeDtypeStruct((ngrid * _TQ, 128), jnp.float32),
    )(tab_t)


def _sc_gather_one(idx, table):
    mesh = plsc.VectorSubcoreMesh(core_axis_name="c", subcore_axis_name="s")

    @functools.partial(
        pl.kernel,
        out_type=jax.ShapeDtypeStruct((_BATCH // 4, 128), jnp.float32),
        mesh=mesh,
        compiler_params=pltpu.CompilerParams(use_tc_tiling_on_sc=True,
                                             needs_layout_passes=False),
        scratch_types=[
            pltpu.VMEM((_BPW,), jnp.int32),
            pltpu.VMEM((_BPW,), jnp.int32),
            pltpu.VMEM((_CH, 128), jnp.float32),
            pltpu.VMEM((_CH, 128), jnp.float32),
            pltpu.VMEM((_BPW // 4, 128), jnp.float32),
            pltpu.SemaphoreType.DMA,
            pltpu.SemaphoreType.DMA,
        ],
    )
    def gather(idx_hbm, tab_hbm, out_hbm,
               idx_v, vrow_v, slab_a, slab_b, out_v, sem_a, sem_b):
        wid = lax.axis_index("s") * _NC + lax.axis_index("c")
        base = pl.multiple_of(wid * _BPW, _BPW)
        base4 = pl.multiple_of(wid * (_BPW // 4), _BPW // 4)
        pltpu.sync_copy(idx_hbm.at[pl.ds(base, _BPW)], idx_v)

        def vrows(k, carry):
            s = pl.ds(k * _L, _L)
            r = idx_v[s]
            vrow_v[s] = lax.shift_left(
                lax.shift_right_logical(r, 13), 11) + lax.bitwise_and(
                    r, _TQ - 1)
            return carry

        lax.fori_loop(0, _BPW // _L, vrows, 0)

        def fire_chunk(c, slab_v, sem):
            s = pl.ds(pl.multiple_of(c * _CH, _CH), _CH)
            return pltpu.async_copy(tab_hbm.at[vrow_v.at[s]], slab_v, sem)

        def extract(slab_v, c):
            for g in range(_CH // _L):
                lrow = lax.iota(jnp.int32, _L) + g * _L
                off = pl.multiple_of(c * _CH + g * _L, _L)
                col0 = lax.bitwise_and(
                    lax.shift_right_logical(idx_v[pl.ds(off, _L)], 11),
                    3) * _EMB
                grow = lrow + c * _CH
                drow = lax.shift_right_logical(grow, 2)
                dcol0 = lax.bitwise_and(grow, 3) * _EMB
                for j in range(_EMB):
                    vals = plsc.load_gather(slab_v, [lrow, col0 + j])
                    plsc.store_scatter(out_v, [drow, dcol0 + j], vals)

        nchunks = _BPW // _CH
        fire_chunk(0, slab_a, sem_a)

        def body(k, _):
            ca = 2 * k
            fire_chunk(ca + 1, slab_b, sem_b)
            pltpu.make_async_copy(
                tab_hbm.at[pl.ds(0, _CH)], slab_a, sem_a).wait()
            extract(slab_a, ca)

            @pl.when(k < nchunks // 2 - 1)
            def _():
                fire_chunk(ca + 2, slab_a, sem_a)

            pltpu.make_async_copy(
                tab_hbm.at[pl.ds(0, _CH)], slab_b, sem_b).wait()
            extract(slab_b, ca + 1)
            return 0

        lax.fori_loop(0, nchunks // 2, body, 0)

        pltpu.sync_copy(out_v, out_hbm.at[pl.ds(base4, _BPW // 4)])

    return gather(idx, table)


def _mlp_body(u_ref, i_ref, w1u_ref, w1i_ref, b1_ref, w2_ref, b2_ref,
              w3_ref, b3_ref, o_ref):
    dn = (((1,), (1,)), ((), ()))
    x1 = lax.dot_general(u_ref[...], w1u_ref[...], dn,
                         preferred_element_type=jnp.float32)
    x1 = x1 + lax.dot_general(i_ref[...], w1i_ref[...], dn,
                              preferred_element_type=jnp.float32)
    x1 = jnp.maximum(x1 + b1_ref[...], 0.0)
    x2 = lax.dot_general(x1, w2_ref[...], dn,
                         preferred_element_type=jnp.float32)
    x2 = jnp.maximum(x2 + b2_ref[...], 0.0)
    z = jnp.sum(x2 * w3_ref[...], axis=1, keepdims=True)
    z = z + b3_ref[0]
    o_ref[...] = 1.0 / (1.0 + jnp.exp(-z))


def _tc_mlp(u_emb, i_emb, W1u, W1i, b1r, W2, b2r, W3, b3):
    grid = (_BATCH // _BT,)
    full = lambda shape: pl.BlockSpec(shape, lambda i: (0, 0))
    return pl.pallas_call(
        _mlp_body,
        grid=grid,
        in_specs=[
            pl.BlockSpec((_BT, _EMB), lambda i: (i, 0)),
            pl.BlockSpec((_BT, _EMB), lambda i: (i, 0)),
            full(W1u.shape),
            full(W1i.shape),
            full(b1r.shape),
            full(W2.shape),
            full(b2r.shape),
            full(W3.shape),
            pl.BlockSpec(memory_space=pltpu.SMEM),
        ],
        out_specs=pl.BlockSpec((_BT, 1), lambda i: (i, 0)),
        out_shape=jax.ShapeDtypeStruct((_BATCH, 1), jnp.float32),
    )(u_emb, i_emb, W1u, W1i, b1r, W2, b2r, W3, b3)


def kernel(user_idx, item_idx, user_table, item_table, W1, b1, W2, b2, W3, b3):
    uidx = user_idx.astype(jnp.int32)
    iidx = item_idx.astype(jnp.int32)
    u128 = _tc_transpose(user_table.T)
    i128 = _tc_transpose(item_table.T)
    u_out = _sc_gather_one(uidx, u128)
    i_out = _sc_gather_one(iidx, i128)
    u_emb = u_out.reshape(_BATCH, _EMB)
    i_emb = i_out.reshape(_BATCH, _EMB)
    W1u = W1[:, :_EMB]
    W1i = W1[:, _EMB:]
    return _tc_mlp(u_emb, i_emb, W1u, W1i,
                   b1.reshape(1, -1), W2, b2.reshape(1, -1),
                   W3, b3)


# speedup vs baseline: 1.6220x; 1.0142x over previous
import functools

import jax
import jax.numpy as jnp
from jax import lax
from jax.experimental import pallas as pl
from jax.experimental.pallas import tpu as pltpu
from jax.experimental.pallas import tpu_sc as plsc

_BATCH = 16384
_EMB = 32
_NC = 2
_NS = 16
_NW = _NC * _NS
_BPW = _BATCH // _NW
_CH = 128
_L = 16

_ROWS = 1000000
_TL = 16384
_TQ = _TL // 4

_BT = 2048


def _tr_body(in_ref, out_ref):
    t = jnp.transpose(in_ref[...], (1, 0))
    for a in range(4):
        out_ref[:, 32 * a:32 * (a + 1)] = lax.slice(
            t, (_TQ * a, 0), (_TQ * (a + 1), _EMB))


def _tc_transpose(tab_t):
    ngrid = (_ROWS + _TL - 1) // _TL
    return pl.pallas_call(
        _tr_body,
        grid=(ngrid,),
        in_specs=[pl.BlockSpec((_EMB, _TL), lambda i: (0, i))],
        out_specs=pl.BlockSpec((_TQ, 128), lambda i: (i, 0)),
        out_shape=jax.ShapeDtypeStruct((ngrid * _TQ, 128), jnp.float32),
    )(tab_t)


def _sc_gather_one(idx, table):
    mesh = plsc.VectorSubcoreMesh(core_axis_name="c", subcore_axis_name="s")

    @functools.partial(
        pl.kernel,
        out_type=jax.ShapeDtypeStruct((_BATCH // 4, 128), jnp.float32),
        mesh=mesh,
        compiler_params=pltpu.CompilerParams(use_tc_tiling_on_sc=True,
                                             needs_layout_passes=False),
        scratch_types=[
            pltpu.VMEM((_BPW,), jnp.int32),
            pltpu.VMEM((_BPW,), jnp.int32),
            pltpu.VMEM((_CH, 128), jnp.float32),
            pltpu.VMEM((_CH, 128), jnp.float32),
            pltpu.VMEM((_BPW // 4, 128), jnp.float32),
            pltpu.SemaphoreType.DMA,
            pltpu.SemaphoreType.DMA,
        ],
    )
    def gather(idx_hbm, tab_hbm, out_hbm,
               idx_v, vrow_v, slab_a, slab_b, out_v, sem_a, sem_b):
        wid = lax.axis_index("s") * _NC + lax.axis_index("c")
        base = pl.multiple_of(wid * _BPW, _BPW)
        base4 = pl.multiple_of(wid * (_BPW // 4), _BPW // 4)
        pltpu.sync_copy(idx_hbm.at[pl.ds(base, _BPW)], idx_v)

        def vrows(k, carry):
            s = pl.ds(k * _L, _L)
            r = idx_v[s]
            vrow_v[s] = lax.shift_left(
                lax.shift_right_logical(r, 14), 12) + lax.bitwise_and(
                    r, _TQ - 1)
            return carry

        lax.fori_loop(0, _BPW // _L, vrows, 0)

        def fire_chunk(c, slab_v, sem):
            s = pl.ds(pl.multiple_of(c * _CH, _CH), _CH)
            return pltpu.async_copy(tab_hbm.at[vrow_v.at[s]], slab_v, sem)

        def extract(slab_v, c):
            for g in range(_CH // _L):
                lrow = lax.iota(jnp.int32, _L) + g * _L
                off = pl.multiple_of(c * _CH + g * _L, _L)
                col0 = lax.bitwise_and(
                    lax.shift_right_logical(idx_v[pl.ds(off, _L)], 12),
                    3) * _EMB
                grow = lrow + c * _CH
                drow = lax.shift_right_logical(grow, 2)
                dcol0 = lax.bitwise_and(grow, 3) * _EMB
                for j in range(_EMB):
                    vals = plsc.load_gather(slab_v, [lrow, col0 + j])
                    plsc.store_scatter(out_v, [drow, dcol0 + j], vals)

        nchunks = _BPW // _CH
        fire_chunk(0, slab_a, sem_a)

        def body(k, _):
            ca = 2 * k
            fire_chunk(ca + 1, slab_b, sem_b)
            pltpu.make_async_copy(
                tab_hbm.at[pl.ds(0, _CH)], slab_a, sem_a).wait()
            extract(slab_a, ca)

            @pl.when(k < nchunks // 2 - 1)
            def _():
                fire_chunk(ca + 2, slab_a, sem_a)

            pltpu.make_async_copy(
                tab_hbm.at[pl.ds(0, _CH)], slab_b, sem_b).wait()
            extract(slab_b, ca + 1)
            return 0

        lax.fori_loop(0, nchunks // 2, body, 0)

        pltpu.sync_copy(out_v, out_hbm.at[pl.ds(base4, _BPW // 4)])

    return gather(idx, table)


def _mlp_body(u_ref, i_ref, w1u_ref, w1i_ref, b1_ref, w2_ref, b2_ref,
              w3_ref, b3_ref, o_ref):
    dn = (((1,), (1,)), ((), ()))
    x1 = lax.dot_general(u_ref[...], w1u_ref[...], dn,
                         preferred_element_type=jnp.float32)
    x1 = x1 + lax.dot_general(i_ref[...], w1i_ref[...], dn,
                              preferred_element_type=jnp.float32)
    x1 = jnp.maximum(x1 + b1_ref[...], 0.0)
    x2 = lax.dot_general(x1, w2_ref[...], dn,
                         preferred_element_type=jnp.float32)
    x2 = jnp.maximum(x2 + b2_ref[...], 0.0)
    z = jnp.sum(x2 * w3_ref[...], axis=1, keepdims=True)
    z = z + b3_ref[0]
    o_ref[...] = 1.0 / (1.0 + jnp.exp(-z))


def _tc_mlp(u_emb, i_emb, W1u, W1i, b1r, W2, b2r, W3, b3):
    grid = (_BATCH // _BT,)
    full = lambda shape: pl.BlockSpec(shape, lambda i: (0, 0))
    return pl.pallas_call(
        _mlp_body,
        grid=grid,
        in_specs=[
            pl.BlockSpec((_BT, _EMB), lambda i: (i, 0)),
            pl.BlockSpec((_BT, _EMB), lambda i: (i, 0)),
            full(W1u.shape),
            full(W1i.shape),
            full(b1r.shape),
            full(W2.shape),
            full(b2r.shape),
            full(W3.shape),
            pl.BlockSpec(memory_space=pltpu.SMEM),
        ],
        out_specs=pl.BlockSpec((_BT, 1), lambda i: (i, 0)),
        out_shape=jax.ShapeDtypeStruct((_BATCH, 1), jnp.float32),
    )(u_emb, i_emb, W1u, W1i, b1r, W2, b2r, W3, b3)


def kernel(user_idx, item_idx, user_table, item_table, W1, b1, W2, b2, W3, b3):
    uidx = user_idx.astype(jnp.int32)
    iidx = item_idx.astype(jnp.int32)
    u128 = _tc_transpose(user_table.T)
    i128 = _tc_transpose(item_table.T)
    u_out = _sc_gather_one(uidx, u128)
    i_out = _sc_gather_one(iidx, i128)
    u_emb = u_out.reshape(_BATCH, _EMB)
    i_emb = i_out.reshape(_BATCH, _EMB)
    W1u = W1[:, :_EMB]
    W1i = W1[:, _EMB:]
    return _tc_mlp(u_emb, i_emb, W1u, W1i,
                   b1.reshape(1, -1), W2, b2.reshape(1, -1),
                   W3, b3)
